# Initial kernel scaffold; baseline (speedup 1.0000x reference)
import jax
import jax.numpy as jnp
from jax.experimental import pallas as pl


def kernel(boxes, scores):
    raise NotImplementedError("write your pallas kernel here")


# speedup vs baseline: 37.2886x; 37.2886x over previous
import jax
import jax.numpy as jnp
from jax import lax
from jax.experimental import pallas as pl
from jax.experimental.pallas import tpu as pltpu
from jax.experimental.pallas import tpu_sc as plsc

N = 20000
G = 128
GRID_WORDS = G * G * 2
BITS_WORDS = 640
IOU_THRESH = 0.001
EMPTY16 = 0xFFFF

_mesh = plsc.VectorSubcoreMesh(core_axis_name="c", subcore_axis_name="s")


def _nms_body(x1_h, y1_h, x2_h, y2_h, ord_h, ginit_h, bits_out,
              x1v, y1v, x2v, y2v, ordv, gridv, bitsv):
    is_worker = (lax.axis_index("c") == 0) & (lax.axis_index("s") == 0)

    @pl.when(is_worker)
    def _():
        pltpu.sync_copy(x1_h, x1v)
        pltpu.sync_copy(y1_h, y1v)
        pltpu.sync_copy(x2_h, x2v)
        pltpu.sync_copy(y2_h, y2v)
        pltpu.sync_copy(ord_h, ordv)
        pltpu.sync_copy(ginit_h, gridv)

        lid = lax.iota(jnp.int32, 16)
        zeros = lid * 0

        def init_bits(j, carry):
            bitsv[pl.ds(j * 16, 16)] = zeros
            return carry

        lax.fori_loop(0, BITS_WORDS // 16, init_bits, 0)

        w0 = lid & 1
        c0 = lax.shift_right_logical(lid, 1)
        dx0 = lax.rem(c0, 3) - 1
        dy0 = lax.div(c0, 3) - 1
        in1 = lid < 2
        dx1 = jnp.where(in1, 1, 0)
        dy1 = jnp.where(in1, 1, 0)
        w1 = jnp.where(in1, lid & 1, 0)

        def iou_suppress(cand_u16, x1i, y1i, x2i, y2i, ai):
            valid = cand_u16 != EMPTY16
            ci = jnp.where(valid, cand_u16, 0)
            x1c = plsc.load_gather(x1v, [ci])
            y1c = plsc.load_gather(y1v, [ci])
            x2c = plsc.load_gather(x2v, [ci])
            y2c = plsc.load_gather(y2v, [ci])
            xx1 = jnp.maximum(x1i, x1c)
            yy1 = jnp.maximum(y1i, y1c)
            xx2 = jnp.minimum(x2i, x2c)
            yy2 = jnp.minimum(y2i, y2c)
            inter = jnp.maximum(xx2 - xx1, 0.0) * jnp.maximum(yy2 - yy1, 0.0)
            ac = (x2c - x1c) * (y2c - y1c)
            iou = inter / (ai + ac - inter)
            return jnp.any(valid & (iou > jnp.float32(IOU_THRESH)))

        def body(i, carry):
            wi = jnp.full((16,), i >> 1, jnp.int32)
            word = plsc.load_gather(ordv, [wi])
            sh = jnp.full((16,), (i & 1) * 16, jnp.int32)
            oi = lax.shift_right_logical(word, sh) & EMPTY16
            x1i = plsc.load_gather(x1v, [oi])
            y1i = plsc.load_gather(y1v, [oi])
            x2i = plsc.load_gather(x2v, [oi])
            y2i = plsc.load_gather(y2v, [oi])
            ai = (x2i - x1i) * (y2i - y1i)
            cx = ((x1i + x2i) * jnp.float32(0.125)).astype(jnp.int32)
            cy = ((y1i + y2i) * jnp.float32(0.125)).astype(jnp.int32)
            cx = jnp.clip(cx, 0, G - 1)
            cy = jnp.clip(cy, 0, G - 1)

            def addr(dx, dy, w):
                nx = jnp.clip(cx + dx, 0, G - 1)
                ny = jnp.clip(cy + dy, 0, G - 1)
                return (ny * G + nx) * 2 + w

            g0 = plsc.load_gather(gridv, [addr(dx0, dy0, w0)])
            g1 = plsc.load_gather(gridv, [addr(dx1, dy1, w1)])
            any_cand = jnp.any((g0 != -1) | (g1 != -1))

            def slow():
                s = jnp.zeros((), jnp.bool_)
                for g in (g0, g1):
                    lo = g & EMPTY16
                    hi = lax.shift_right_logical(g, 16) & EMPTY16
                    s |= iou_suppress(lo, x1i, y1i, x2i, y2i, ai)
                    s |= iou_suppress(hi, x1i, y1i, x2i, y2i, ai)
                return s

            sup = lax.cond(any_cand, slow, lambda: jnp.zeros((), jnp.bool_))

            @pl.when(jnp.logical_not(sup))
            def _():
                base2 = (cy * G + cx) * 2
                w0v = plsc.load_gather(gridv, [base2])
                w1v = plsc.load_gather(gridv, [base2 + 1])
                lo0 = w0v & EMPTY16
                hi0 = lax.shift_right_logical(w0v, 16) & EMPTY16
                lo1 = w1v & EMPTY16
                hi1 = lax.shift_right_logical(w1v, 16) & EMPTY16
                e0 = lo0 == EMPTY16
                e1 = (~e0) & (hi0 == EMPTY16)
                e2 = (~e0) & (~e1) & (lo1 == EMPTY16)
                e3 = (~e0) & (~e1) & (~e2) & (hi1 == EMPTY16)
                hi_mask = jnp.int32(-65536)
                lo_mask = jnp.int32(65535)
                oish = lax.shift_left(oi, 16)
                neww0 = jnp.where(e0, (w0v & hi_mask) | oi,
                                  jnp.where(e1, (w0v & lo_mask) | oish, w0v))
                neww1 = jnp.where(e2, (w1v & hi_mask) | oi,
                                  jnp.where(e3, (w1v & lo_mask) | oish, w1v))
                vals = jnp.where(lid == 0, neww0, neww1)
                plsc.store_scatter(gridv, [base2 + lid], vals, mask=lid < 2)
                bit = lax.shift_left(jnp.int32(1), oi & 31)
                plsc.addupdate_scatter(
                    bitsv, [lax.shift_right_logical(oi, 5)], bit, mask=lid == 0)

            return carry

        lax.fori_loop(0, N, body, 0)
        pltpu.sync_copy(bitsv, bits_out)


_nms_call = pl.kernel(
    _nms_body,
    out_type=jax.ShapeDtypeStruct((BITS_WORDS,), jnp.int32),
    mesh=_mesh,
    compiler_params=pltpu.CompilerParams(needs_layout_passes=False),
    scratch_types=[
        pltpu.VMEM((N,), jnp.float32),
        pltpu.VMEM((N,), jnp.float32),
        pltpu.VMEM((N,), jnp.float32),
        pltpu.VMEM((N,), jnp.float32),
        pltpu.VMEM((N // 2,), jnp.int32),
        pltpu.VMEM((GRID_WORDS,), jnp.int32),
        pltpu.VMEM((BITS_WORDS,), jnp.int32),
    ],
)


def kernel(boxes, scores):
    order = jnp.argsort(-scores).astype(jnp.int32)
    ord_packed = order[0::2] | lax.shift_left(order[1::2], 16)
    x1 = boxes[:, 0]
    y1 = boxes[:, 1]
    x2 = boxes[:, 2]
    y2 = boxes[:, 3]
    grid_init = jnp.full((GRID_WORDS,), -1, jnp.int32)
    bits = _nms_call(x1, y1, x2, y2, ord_packed, grid_init)
    idx = jnp.arange(N, dtype=jnp.int32)
    keep = lax.shift_right_logical(bits[idx >> 5], idx & 31) & 1
    m = keep.astype(boxes.dtype)
    return jnp.concatenate([boxes * m[:, None], (scores * m)[:, None]], axis=1)


# speedup vs baseline: 40.1561x; 1.0769x over previous
import jax
import jax.numpy as jnp
from jax import lax
from jax.experimental import pallas as pl
from jax.experimental.pallas import tpu as pltpu
from jax.experimental.pallas import tpu_sc as plsc

N = 20000
G = 128
GRID_WORDS = G * G * 2
BITS_WORDS = 640
IOU_THRESH = 0.001
EMPTY16 = 0xFFFF

_mesh = plsc.VectorSubcoreMesh(core_axis_name="c", subcore_axis_name="s")


def _nms_body(x1_h, y1_h, x2_h, y2_h, ord_h, ginit_h, bits_out,
              x1v, y1v, x2v, y2v, ordv, gridv, bitsv):
    is_worker = (lax.axis_index("c") == 0) & (lax.axis_index("s") == 0)

    @pl.when(is_worker)
    def _():
        pltpu.sync_copy(x1_h, x1v)
        pltpu.sync_copy(y1_h, y1v)
        pltpu.sync_copy(x2_h, x2v)
        pltpu.sync_copy(y2_h, y2v)
        pltpu.sync_copy(ord_h, ordv)
        pltpu.sync_copy(ginit_h, gridv)

        lid = lax.iota(jnp.int32, 16)
        zeros = lid * 0

        def init_bits(j, carry):
            bitsv[pl.ds(j * 16, 16)] = zeros
            return carry

        lax.fori_loop(0, BITS_WORDS // 16, init_bits, 0)

        w0 = lid & 1
        c0 = lax.shift_right_logical(lid, 1)
        dx0 = lax.rem(c0, 3) - 1
        dy0 = lax.div(c0, 3) - 1
        in1 = lid < 2
        dx1 = jnp.where(in1, 1, 0)
        dy1 = jnp.where(in1, 1, 0)
        w1 = jnp.where(in1, lid & 1, 0)

        def iou_suppress(cand_u16, x1i, y1i, x2i, y2i, ai):
            valid = cand_u16 != EMPTY16
            ci = jnp.where(valid, cand_u16, 0)
            x1c = plsc.load_gather(x1v, [ci])
            y1c = plsc.load_gather(y1v, [ci])
            x2c = plsc.load_gather(x2v, [ci])
            y2c = plsc.load_gather(y2v, [ci])
            xx1 = jnp.maximum(x1i, x1c)
            yy1 = jnp.maximum(y1i, y1c)
            xx2 = jnp.minimum(x2i, x2c)
            yy2 = jnp.minimum(y2i, y2c)
            inter = jnp.maximum(xx2 - xx1, 0.0) * jnp.maximum(yy2 - yy1, 0.0)
            ac = (x2c - x1c) * (y2c - y1c)
            iou = inter / (ai + ac - inter)
            return jnp.any(valid & (iou > jnp.float32(IOU_THRESH)))

        def broadcast_lane(vec, k):
            return jnp.take_along_axis(vec, jnp.full((16,), k, jnp.int32),
                                       axis=0)

        def outer(jb, carry):
            wblk = ordv[pl.ds(jb * 16, 16)]

            def body(k, carry):
                word = broadcast_lane(wblk, k >> 1)
                sh = jnp.full((16,), (k & 1) * 16, jnp.int32)
                oi = lax.shift_right_logical(word, sh) & EMPTY16
                x1i = plsc.load_gather(x1v, [oi])
                y1i = plsc.load_gather(y1v, [oi])
                x2i = plsc.load_gather(x2v, [oi])
                y2i = plsc.load_gather(y2v, [oi])
                ai = (x2i - x1i) * (y2i - y1i)
                cx = ((x1i + x2i) * jnp.float32(0.125)).astype(jnp.int32)
                cy = ((y1i + y2i) * jnp.float32(0.125)).astype(jnp.int32)
                cx = jnp.clip(cx, 0, G - 1)
                cy = jnp.clip(cy, 0, G - 1)

                def addr(dx, dy, w):
                    nx = jnp.clip(cx + dx, 0, G - 1)
                    ny = jnp.clip(cy + dy, 0, G - 1)
                    return (ny * G + nx) * 2 + w

                g0 = plsc.load_gather(gridv, [addr(dx0, dy0, w0)])
                g1 = plsc.load_gather(gridv, [addr(dx1, dy1, w1)])
                any_cand = jnp.any((g0 != -1) | (g1 != -1))

                def slow():
                    s = jnp.zeros((), jnp.bool_)
                    for g in (g0, g1):
                        lo = g & EMPTY16
                        hi = lax.shift_right_logical(g, 16) & EMPTY16
                        s |= iou_suppress(lo, x1i, y1i, x2i, y2i, ai)
                        s |= iou_suppress(hi, x1i, y1i, x2i, y2i, ai)
                    return s

                sup = lax.cond(any_cand, slow,
                               lambda: jnp.zeros((), jnp.bool_))

                @pl.when(jnp.logical_not(sup))
                def _():
                    w0v = broadcast_lane(g0, 8)
                    w1v = broadcast_lane(g0, 9)
                    base2 = (cy * G + cx) * 2
                    lo0 = w0v & EMPTY16
                    hi0 = lax.shift_right_logical(w0v, 16) & EMPTY16
                    lo1 = w1v & EMPTY16
                    hi1 = lax.shift_right_logical(w1v, 16) & EMPTY16
                    e0 = lo0 == EMPTY16
                    e1 = (~e0) & (hi0 == EMPTY16)
                    e2 = (~e0) & (~e1) & (lo1 == EMPTY16)
                    e3 = (~e0) & (~e1) & (~e2) & (hi1 == EMPTY16)
                    hi_mask = jnp.int32(-65536)
                    lo_mask = jnp.int32(65535)
                    oish = lax.shift_left(oi, 16)
                    neww0 = jnp.where(e0, (w0v & hi_mask) | oi,
                                      jnp.where(e1, (w0v & lo_mask) | oish,
                                                w0v))
                    neww1 = jnp.where(e2, (w1v & hi_mask) | oi,
                                      jnp.where(e3, (w1v & lo_mask) | oish,
                                                w1v))
                    vals = jnp.where(lid == 0, neww0, neww1)
                    plsc.store_scatter(gridv, [base2 + lid], vals,
                                       mask=lid < 2)
                    bit = lax.shift_left(jnp.int32(1), oi & 31)
                    plsc.addupdate_scatter(
                        bitsv, [lax.shift_right_logical(oi, 5)], bit,
                        mask=lid == 0)

                return carry

            lax.fori_loop(0, 32, body, 0)
            return carry

        lax.fori_loop(0, N // 32, outer, 0)
        pltpu.sync_copy(bitsv, bits_out)


_nms_call = pl.kernel(
    _nms_body,
    out_type=jax.ShapeDtypeStruct((BITS_WORDS,), jnp.int32),
    mesh=_mesh,
    compiler_params=pltpu.CompilerParams(needs_layout_passes=False),
    scratch_types=[
        pltpu.VMEM((N,), jnp.float32),
        pltpu.VMEM((N,), jnp.float32),
        pltpu.VMEM((N,), jnp.float32),
        pltpu.VMEM((N,), jnp.float32),
        pltpu.VMEM((N // 2,), jnp.int32),
        pltpu.VMEM((GRID_WORDS,), jnp.int32),
        pltpu.VMEM((BITS_WORDS,), jnp.int32),
    ],
)


def kernel(boxes, scores):
    order = jnp.argsort(-scores).astype(jnp.int32)
    ord_packed = order[0::2] | lax.shift_left(order[1::2], 16)
    x1 = boxes[:, 0]
    y1 = boxes[:, 1]
    x2 = boxes[:, 2]
    y2 = boxes[:, 3]
    grid_init = jnp.full((GRID_WORDS,), -1, jnp.int32)
    bits = _nms_call(x1, y1, x2, y2, ord_packed, grid_init)
    idx = jnp.arange(N, dtype=jnp.int32)
    keep = lax.shift_right_logical(bits[idx >> 5], idx & 31) & 1
    m = keep.astype(boxes.dtype)
    return jnp.concatenate([boxes * m[:, None], (scores * m)[:, None]], axis=1)


# speedup vs baseline: 67.8099x; 1.6887x over previous
import jax
import jax.numpy as jnp
from jax import lax
from jax.experimental import pallas as pl
from jax.experimental.pallas import tpu as pltpu
from jax.experimental.pallas import tpu_sc as plsc

N = 20000
G = 128
GP = G + 2
GRID_WORDS = (GP * GP * 2 + 15) // 16 * 16
BITS_WORDS = 640
IOU_THRESH = 0.001
EMPTY16 = 0xFFFF

_mesh = plsc.VectorSubcoreMesh(core_axis_name="c", subcore_axis_name="s")


def _nms_body(x1_h, y1_h, x2_h, y2_h, ord_h, ginit_h, bits_out,
              x1v, y1v, x2v, y2v, ordv, gridv, bitsv):
    is_worker = (lax.axis_index("c") == 0) & (lax.axis_index("s") == 0)

    @pl.when(is_worker)
    def _():
        pltpu.sync_copy(x1_h, x1v)
        pltpu.sync_copy(y1_h, y1v)
        pltpu.sync_copy(x2_h, x2v)
        pltpu.sync_copy(y2_h, y2v)
        pltpu.sync_copy(ord_h, ordv)
        pltpu.sync_copy(ginit_h, gridv)

        lid = lax.iota(jnp.int32, 16)
        zeros = lid * 0

        def init_bits(j, carry):
            bitsv[pl.ds(j * 16, 16)] = zeros
            return carry

        lax.fori_loop(0, BITS_WORDS // 16, init_bits, 0)

        w0 = lid & 1
        c0 = lax.shift_right_logical(lid, 1)
        off0 = ((lax.div(c0, 3) - 1) * GP + lax.rem(c0, 3) - 1) * 2 + w0
        in1 = lid < 2
        off1 = jnp.where(in1, (GP + 1) * 2 + (lid & 1), 0)

        def iou_mask(cand_u16, x1i, y1i, x2i, y2i, ai):
            valid = cand_u16 != EMPTY16
            ci = jnp.where(valid, cand_u16, 0)
            x1c = plsc.load_gather(x1v, [ci])
            y1c = plsc.load_gather(y1v, [ci])
            x2c = plsc.load_gather(x2v, [ci])
            y2c = plsc.load_gather(y2v, [ci])
            xx1 = jnp.maximum(x1i, x1c)
            yy1 = jnp.maximum(y1i, y1c)
            xx2 = jnp.minimum(x2i, x2c)
            yy2 = jnp.minimum(y2i, y2c)
            inter = jnp.maximum(xx2 - xx1, 0.0) * jnp.maximum(yy2 - yy1, 0.0)
            ac = (x2c - x1c) * (y2c - y1c)
            iou = inter / (ai + ac - inter)
            return valid & (iou > jnp.float32(IOU_THRESH))

        def broadcast_lane(vec, k):
            return jnp.take_along_axis(vec, jnp.full((16,), k, jnp.int32),
                                       axis=0)

        def outer(jb, carry):
            wblk = ordv[pl.ds(jb * 16, 16)]

            def body(k, carry):
                word = broadcast_lane(wblk, k >> 1)
                sh = jnp.full((16,), (k & 1) * 16, jnp.int32)
                oi = lax.shift_right_logical(word, sh) & EMPTY16
                x1i = plsc.load_gather(x1v, [oi])
                y1i = plsc.load_gather(y1v, [oi])
                x2i = plsc.load_gather(x2v, [oi])
                y2i = plsc.load_gather(y2v, [oi])
                ai = (x2i - x1i) * (y2i - y1i)
                cx = ((x1i + x2i) * jnp.float32(0.125)).astype(jnp.int32)
                cy = ((y1i + y2i) * jnp.float32(0.125)).astype(jnp.int32)
                cx = jnp.clip(cx, 0, G - 1)
                cy = jnp.clip(cy, 0, G - 1)
                base2 = ((cy + 1) * GP + (cx + 1)) * 2

                g0 = plsc.load_gather(gridv, [base2 + off0])
                g1 = plsc.load_gather(gridv, [base2 + off1])

                sup = (
                    iou_mask(g0 & EMPTY16, x1i, y1i, x2i, y2i, ai)
                    | iou_mask(lax.shift_right_logical(g0, 16) & EMPTY16,
                               x1i, y1i, x2i, y2i, ai)
                    | iou_mask(g1 & EMPTY16, x1i, y1i, x2i, y2i, ai)
                    | iou_mask(lax.shift_right_logical(g1, 16) & EMPTY16,
                               x1i, y1i, x2i, y2i, ai)
                )
                keep = plsc.all_reduce_population_count(sup) == 0

                w0v = broadcast_lane(g0, 8)
                w1v = broadcast_lane(g0, 9)
                lo0 = w0v & EMPTY16
                hi0 = lax.shift_right_logical(w0v, 16) & EMPTY16
                lo1 = w1v & EMPTY16
                e0 = lo0 == EMPTY16
                e1 = (~e0) & (hi0 == EMPTY16)
                e2 = (~e0) & (~e1) & (lo1 == EMPTY16)
                e3 = (~e0) & (~e1) & (~e2)
                hi_mask = jnp.int32(-65536)
                lo_mask = jnp.int32(65535)
                oish = lax.shift_left(oi, 16)
                neww0 = jnp.where(e0, (w0v & hi_mask) | oi,
                                  jnp.where(e1, (w0v & lo_mask) | oish, w0v))
                neww1 = jnp.where(e2, (w1v & hi_mask) | oi,
                                  jnp.where(e3, (w1v & lo_mask) | oish, w1v))
                vals = jnp.where(lid == 0, neww0, neww1)
                plsc.store_scatter(gridv, [base2 + lid], vals,
                                   mask=keep & (lid < 2))
                bit = lax.shift_left(jnp.int32(1), oi & 31)
                plsc.addupdate_scatter(
                    bitsv, [lax.shift_right_logical(oi, 5)], bit,
                    mask=keep & (lid == 0))

                return carry

            lax.fori_loop(0, 32, body, 0)
            return carry

        lax.fori_loop(0, N // 32, outer, 0)
        pltpu.sync_copy(bitsv, bits_out)


_nms_call = pl.kernel(
    _nms_body,
    out_type=jax.ShapeDtypeStruct((BITS_WORDS,), jnp.int32),
    mesh=_mesh,
    compiler_params=pltpu.CompilerParams(needs_layout_passes=False),
    scratch_types=[
        pltpu.VMEM((N,), jnp.float32),
        pltpu.VMEM((N,), jnp.float32),
        pltpu.VMEM((N,), jnp.float32),
        pltpu.VMEM((N,), jnp.float32),
        pltpu.VMEM((N // 2,), jnp.int32),
        pltpu.VMEM((GRID_WORDS,), jnp.int32),
        pltpu.VMEM((BITS_WORDS,), jnp.int32),
    ],
)


def kernel(boxes, scores):
    order = jnp.argsort(-scores).astype(jnp.int32)
    ord_packed = order[0::2] | lax.shift_left(order[1::2], 16)
    x1 = boxes[:, 0]
    y1 = boxes[:, 1]
    x2 = boxes[:, 2]
    y2 = boxes[:, 3]
    grid_init = jnp.full((GRID_WORDS,), -1, jnp.int32)
    bits = _nms_call(x1, y1, x2, y2, ord_packed, grid_init)
    idx = jnp.arange(N, dtype=jnp.int32)
    keep = lax.shift_right_logical(bits[idx >> 5], idx & 31) & 1
    m = keep.astype(boxes.dtype)
    return jnp.concatenate([boxes * m[:, None], (scores * m)[:, None]], axis=1)


# speedup vs baseline: 78.4721x; 1.1572x over previous
import jax
import jax.numpy as jnp
from jax import lax
from jax.experimental import pallas as pl
from jax.experimental.pallas import tpu as pltpu
from jax.experimental.pallas import tpu_sc as plsc

N = 20000
G = 128
GP = G + 2
GRID_WORDS = (GP * GP * 2 + 15) // 16 * 16
BITS_WORDS = 640
IOU_THRESH = 0.001
EMPTY16 = 0xFFFF

_mesh = plsc.VectorSubcoreMesh(core_axis_name="c", subcore_axis_name="s")


def _nms_body(x1_h, y1_h, x2_h, y2_h, ginit_h, bits_out,
              x1v, y1v, x2v, y2v, gridv, bitsv):
    is_worker = (lax.axis_index("c") == 0) & (lax.axis_index("s") == 0)

    @pl.when(is_worker)
    def _():
        pltpu.sync_copy(x1_h, x1v)
        pltpu.sync_copy(y1_h, y1v)
        pltpu.sync_copy(x2_h, x2v)
        pltpu.sync_copy(y2_h, y2v)
        pltpu.sync_copy(ginit_h, gridv)

        lid = lax.iota(jnp.int32, 16)
        zeros = lid * 0

        def init_bits(j, carry):
            bitsv[pl.ds(j * 16, 16)] = zeros
            return carry

        lax.fori_loop(0, BITS_WORDS // 16, init_bits, 0)

        w0 = lid & 1
        c0 = lax.shift_right_logical(lid, 1)
        off0 = ((lax.div(c0, 3) - 1) * GP + lax.rem(c0, 3) - 1) * 2 + w0
        in1 = lid < 2
        off1 = jnp.where(in1, (GP + 1) * 2 + (lid & 1), 0)
        qidx = lax.shift_right_logical(lid, 1)
        shhalf = lax.shift_left(lid & 1, 4)

        def lanes(vec, idx):
            return jnp.take_along_axis(vec, idx, axis=0)

        def broadcast_lane(vec, k):
            return lanes(vec, jnp.full((16,), k, jnp.int32))

        def iou_mask(cand_u16, x1i, y1i, x2i, y2i, ai):
            valid = cand_u16 != EMPTY16
            ci = jnp.where(valid, cand_u16, 0)
            x1c = plsc.load_gather(x1v, [ci])
            y1c = plsc.load_gather(y1v, [ci])
            x2c = plsc.load_gather(x2v, [ci])
            y2c = plsc.load_gather(y2v, [ci])
            xx1 = jnp.maximum(x1i, x1c)
            yy1 = jnp.maximum(y1i, y1c)
            xx2 = jnp.minimum(x2i, x2c)
            yy2 = jnp.minimum(y2i, y2c)
            inter = jnp.maximum(xx2 - xx1, 0.0) * jnp.maximum(yy2 - yy1, 0.0)
            ac = (x2c - x1c) * (y2c - y1c)
            iou = inter / (ai + ac - inter)
            return valid & (iou > jnp.float32(IOU_THRESH))

        def outer(jb, carry):
            base = jb * 16
            x1blk = x1v[pl.ds(base, 16)]
            y1blk = y1v[pl.ds(base, 16)]
            x2blk = x2v[pl.ds(base, 16)]
            y2blk = y2v[pl.ds(base, 16)]

            def body(k, carry):
                x1i = broadcast_lane(x1blk, k)
                y1i = broadcast_lane(y1blk, k)
                x2i = broadcast_lane(x2blk, k)
                y2i = broadcast_lane(y2blk, k)
                iv = jnp.full((16,), base + k, jnp.int32)
                ai = (x2i - x1i) * (y2i - y1i)
                cx = ((x1i + x2i) * jnp.float32(0.125)).astype(jnp.int32)
                cy = ((y1i + y2i) * jnp.float32(0.125)).astype(jnp.int32)
                cx = jnp.clip(cx, 0, G - 1)
                cy = jnp.clip(cy, 0, G - 1)
                base2 = ((cy + 1) * GP + (cx + 1)) * 2

                g0 = plsc.load_gather(gridv, [base2 + off0])
                g1 = plsc.load_gather(gridv, [base2 + off1])

                def slots(g, wbase):
                    w = lanes(g, wbase)
                    return lax.shift_right_logical(w, shhalf) & EMPTY16

                sup = (
                    iou_mask(slots(g0, qidx), x1i, y1i, x2i, y2i, ai)
                    | iou_mask(slots(g0, qidx + 8), x1i, y1i, x2i, y2i, ai)
                    | iou_mask(slots(g1, qidx), x1i, y1i, x2i, y2i, ai)
                )
                keep = plsc.all_reduce_population_count(sup) == 0

                w0v = broadcast_lane(g0, 8)
                w1v = broadcast_lane(g0, 9)
                lo0 = w0v & EMPTY16
                hi0 = lax.shift_right_logical(w0v, 16) & EMPTY16
                lo1 = w1v & EMPTY16
                e0 = lo0 == EMPTY16
                e1 = (~e0) & (hi0 == EMPTY16)
                e2 = (~e0) & (~e1) & (lo1 == EMPTY16)
                e3 = (~e0) & (~e1) & (~e2)
                hi_mask = jnp.int32(-65536)
                lo_mask = jnp.int32(65535)
                ish = lax.shift_left(iv, 16)
                neww0 = jnp.where(e0, (w0v & hi_mask) | iv,
                                  jnp.where(e1, (w0v & lo_mask) | ish, w0v))
                neww1 = jnp.where(e2, (w1v & hi_mask) | iv,
                                  jnp.where(e3, (w1v & lo_mask) | ish, w1v))
                vals = jnp.where(lid == 0, neww0, neww1)
                plsc.store_scatter(gridv, [base2 + lid], vals,
                                   mask=keep & (lid < 2))
                bit = lax.shift_left(jnp.int32(1), iv & 31)
                plsc.addupdate_scatter(
                    bitsv, [lax.shift_right_logical(iv, 5)], bit,
                    mask=keep & (lid == 0))

                return carry

            lax.fori_loop(0, 16, body, 0)
            return carry

        lax.fori_loop(0, N // 16, outer, 0)
        pltpu.sync_copy(bitsv, bits_out)


_nms_call = pl.kernel(
    _nms_body,
    out_type=jax.ShapeDtypeStruct((BITS_WORDS,), jnp.int32),
    mesh=_mesh,
    compiler_params=pltpu.CompilerParams(needs_layout_passes=False),
    scratch_types=[
        pltpu.VMEM((N,), jnp.float32),
        pltpu.VMEM((N,), jnp.float32),
        pltpu.VMEM((N,), jnp.float32),
        pltpu.VMEM((N,), jnp.float32),
        pltpu.VMEM((GRID_WORDS,), jnp.int32),
        pltpu.VMEM((BITS_WORDS,), jnp.int32),
    ],
)


def kernel(boxes, scores):
    order = jnp.argsort(-scores)
    b = boxes[order]
    grid_init = jnp.full((GRID_WORDS,), -1, jnp.int32)
    bits = _nms_call(b[:, 0], b[:, 1], b[:, 2], b[:, 3], grid_init)
    idx = jnp.arange(N, dtype=jnp.int32)
    keep_sorted = lax.shift_right_logical(bits[idx >> 5], idx & 31) & 1
    keep = jnp.zeros((N,), jnp.int32).at[order].set(keep_sorted)
    m = keep.astype(boxes.dtype)
    return jnp.concatenate([boxes * m[:, None], (scores * m)[:, None]], axis=1)


# speedup vs baseline: 84.4789x; 1.0765x over previous
import jax
import jax.numpy as jnp
from jax import lax
from jax.experimental import pallas as pl
from jax.experimental.pallas import tpu as pltpu
from jax.experimental.pallas import tpu_sc as plsc

N = 20000
NPAD = N + 16
G = 128
GP = G + 2
GRID_WORDS = (GP * GP * 2 + 15) // 16 * 16
BITS_WORDS = 640
IOU_THRESH = 0.001
DUMMY = N
DUMMY_WORD = DUMMY | (DUMMY << 16)

_mesh = plsc.VectorSubcoreMesh(core_axis_name="c", subcore_axis_name="s")


def _nms_body(x1_h, y1_h, x2_h, y2_h, ginit_h, bits_out,
              x1v, y1v, x2v, y2v, gridv, bitsv):
    is_worker = (lax.axis_index("c") == 0) & (lax.axis_index("s") == 0)

    @pl.when(is_worker)
    def _():
        pltpu.sync_copy(x1_h, x1v)
        pltpu.sync_copy(y1_h, y1v)
        pltpu.sync_copy(x2_h, x2v)
        pltpu.sync_copy(y2_h, y2v)
        pltpu.sync_copy(ginit_h, gridv)

        lid = lax.iota(jnp.int32, 16)
        zeros = lid * 0

        def init_bits(j, carry):
            bitsv[pl.ds(j * 16, 16)] = zeros
            return carry

        lax.fori_loop(0, BITS_WORDS // 16, init_bits, 0)

        w0 = lid & 1
        c0 = lax.shift_right_logical(lid, 1)
        off0 = ((lax.div(c0, 3) - 1) * GP + lax.rem(c0, 3) - 1) * 2 + w0
        in1 = lid < 2
        off1 = jnp.where(in1, (GP + 1) * 2 + (lid & 1), 0)
        qidx = lax.shift_right_logical(lid, 1)
        shhalf = lax.shift_left(lid & 1, 4)

        def lanes(vec, idx):
            return jnp.take_along_axis(vec, idx, axis=0)

        def broadcast_lane(vec, k):
            return lanes(vec, jnp.full((16,), k, jnp.int32))

        def iou_mask(ci, x1i, y1i, x2i, y2i, ai):
            x1c = plsc.load_gather(x1v, [ci])
            y1c = plsc.load_gather(y1v, [ci])
            x2c = plsc.load_gather(x2v, [ci])
            y2c = plsc.load_gather(y2v, [ci])
            xx1 = jnp.maximum(x1i, x1c)
            yy1 = jnp.maximum(y1i, y1c)
            xx2 = jnp.minimum(x2i, x2c)
            yy2 = jnp.minimum(y2i, y2c)
            inter = jnp.maximum(xx2 - xx1, 0.0) * jnp.maximum(yy2 - yy1, 0.0)
            ac = (x2c - x1c) * (y2c - y1c)
            iou = inter / (ai + ac - inter)
            return iou > jnp.float32(IOU_THRESH)

        def outer(jb, carry):
            base = jb * 16
            x1blk = x1v[pl.ds(base, 16)]
            y1blk = y1v[pl.ds(base, 16)]
            x2blk = x2v[pl.ds(base, 16)]
            y2blk = y2v[pl.ds(base, 16)]
            iblk = base + lid
            aiblk = (x2blk - x1blk) * (y2blk - y1blk)
            cxblk = ((x1blk + x2blk) * jnp.float32(0.125)).astype(jnp.int32)
            cyblk = ((y1blk + y2blk) * jnp.float32(0.125)).astype(jnp.int32)
            cxblk = jnp.clip(cxblk, 0, G - 1)
            cyblk = jnp.clip(cyblk, 0, G - 1)
            b2blk = ((cyblk + 1) * GP + (cxblk + 1)) * 2
            bitblk = lax.shift_left(jnp.int32(1), iblk & 31)
            widblk = lax.shift_right_logical(iblk, 5)
            ishblk = lax.shift_left(iblk, 16)

            def body(k, carry):
                x1i = broadcast_lane(x1blk, k)
                y1i = broadcast_lane(y1blk, k)
                x2i = broadcast_lane(x2blk, k)
                y2i = broadcast_lane(y2blk, k)
                ai = broadcast_lane(aiblk, k)
                base2 = broadcast_lane(b2blk, k)

                g0 = plsc.load_gather(gridv, [base2 + off0])
                g1 = plsc.load_gather(gridv, [base2 + off1])

                def slots(g, wbase):
                    w = lanes(g, wbase)
                    return lax.shift_right_logical(w, shhalf) & 0xFFFF

                sup = (
                    iou_mask(slots(g0, qidx), x1i, y1i, x2i, y2i, ai)
                    | iou_mask(slots(g0, qidx + 8), x1i, y1i, x2i, y2i, ai)
                    | iou_mask(slots(g1, qidx), x1i, y1i, x2i, y2i, ai)
                )
                keep = plsc.all_reduce_population_count(sup) == 0

                w0v = broadcast_lane(g0, 8)
                w1v = broadcast_lane(g0, 9)
                lo0 = w0v & 0xFFFF
                hi0 = lax.shift_right_logical(w0v, 16) & 0xFFFF
                lo1 = w1v & 0xFFFF
                e0 = lo0 == DUMMY
                e1 = (~e0) & (hi0 == DUMMY)
                e2 = (~e0) & (~e1) & (lo1 == DUMMY)
                e3 = (~e0) & (~e1) & (~e2)
                hi_mask = jnp.int32(-65536)
                lo_mask = jnp.int32(65535)
                iv = broadcast_lane(iblk, k)
                ish = broadcast_lane(ishblk, k)
                neww0 = jnp.where(e0, (w0v & hi_mask) | iv,
                                  jnp.where(e1, (w0v & lo_mask) | ish, w0v))
                neww1 = jnp.where(e2, (w1v & hi_mask) | iv,
                                  jnp.where(e3, (w1v & lo_mask) | ish, w1v))
                vals = jnp.where(lid == 0, neww0, neww1)
                plsc.store_scatter(gridv, [base2 + lid], vals,
                                   mask=keep & (lid < 2))
                plsc.addupdate_scatter(
                    bitsv, [broadcast_lane(widblk, k)],
                    broadcast_lane(bitblk, k), mask=keep & (lid == 0))

                return carry

            lax.fori_loop(0, 16, body, 0)
            return carry

        lax.fori_loop(0, N // 16, outer, 0)
        pltpu.sync_copy(bitsv, bits_out)


_nms_call = pl.kernel(
    _nms_body,
    out_type=jax.ShapeDtypeStruct((BITS_WORDS,), jnp.int32),
    mesh=_mesh,
    compiler_params=pltpu.CompilerParams(needs_layout_passes=False),
    scratch_types=[
        pltpu.VMEM((NPAD,), jnp.float32),
        pltpu.VMEM((NPAD,), jnp.float32),
        pltpu.VMEM((NPAD,), jnp.float32),
        pltpu.VMEM((NPAD,), jnp.float32),
        pltpu.VMEM((GRID_WORDS,), jnp.int32),
        pltpu.VMEM((BITS_WORDS,), jnp.int32),
    ],
)


def kernel(boxes, scores):
    order = jnp.argsort(-scores)
    b = boxes[order]
    far = jnp.full((16,), 2e6, boxes.dtype)
    cols = [jnp.concatenate([b[:, c], far + (3.0 if c >= 2 else 0.0)])
            for c in range(4)]
    grid_init = jnp.full((GRID_WORDS,), DUMMY_WORD, jnp.int32)
    bits = _nms_call(cols[0], cols[1], cols[2], cols[3], grid_init)
    idx = jnp.arange(N, dtype=jnp.int32)
    keep_sorted = lax.shift_right_logical(bits[idx >> 5], idx & 31) & 1
    keep = jnp.zeros((N,), jnp.int32).at[order].set(keep_sorted)
    m = keep.astype(boxes.dtype)
    return jnp.concatenate([boxes * m[:, None], (scores * m)[:, None]], axis=1)
